# Initial kernel scaffold; baseline (speedup 1.0000x reference)
import jax
import jax.numpy as jnp
from jax.experimental import pallas as pl


def kernel(x, edge_index, edge_attr, batch, init_w0, root_w0, bias0, init_w1, root_w1, bias1, head_w1, head_b1, head_w2, head_b2):
    raise NotImplementedError("write your pallas kernel here")


# speedup vs baseline: 1.0409x; 1.0409x over previous
import functools

import jax
import jax.numpy as jnp
from jax import lax
from jax.experimental import pallas as pl
from jax.experimental.pallas import tpu as pltpu

_G = 128


def _mm2_body(x_ref, w1_ref, w2_ref, o1_ref, o2_ref):
    x = x_ref[...]
    o1_ref[...] = jnp.dot(x, w1_ref[...], preferred_element_type=jnp.float32)
    o2_ref[...] = jnp.dot(x, w2_ref[...], preferred_element_type=jnp.float32)


def _mm2(x, w1, w2, bn=2000):
    n, f = x.shape
    h = w1.shape[1]
    return pl.pallas_call(
        _mm2_body,
        grid=(n // bn,),
        in_specs=[pl.BlockSpec((bn, f), lambda i: (i, 0)),
                  pl.BlockSpec((f, h), lambda i: (0, 0)),
                  pl.BlockSpec((f, h), lambda i: (0, 0))],
        out_specs=[pl.BlockSpec((bn, h), lambda i: (i, 0)),
                   pl.BlockSpec((bn, h), lambda i: (i, 0))],
        out_shape=[jax.ShapeDtypeStruct((n, h), jnp.float32),
                   jax.ShapeDtypeStruct((n, h), jnp.float32)],
    )(x, w1, w2)


def _fused_mm2_body(agg_ref, r_ref, b_ref, w1_ref, w2_ref, o1_ref, o2_ref):
    h = jnp.maximum(agg_ref[...] + r_ref[...] + b_ref[...], 0.0)
    o1_ref[...] = jnp.dot(h, w1_ref[...], preferred_element_type=jnp.float32)
    o2_ref[...] = jnp.dot(h, w2_ref[...], preferred_element_type=jnp.float32)


def _fused_mm2(agg, r, bias, w1, w2, bn=2000):
    n, f = agg.shape
    h = w1.shape[1]
    return pl.pallas_call(
        _fused_mm2_body,
        grid=(n // bn,),
        in_specs=[pl.BlockSpec((bn, f), lambda i: (i, 0)),
                  pl.BlockSpec((bn, f), lambda i: (i, 0)),
                  pl.BlockSpec((1, f), lambda i: (0, 0)),
                  pl.BlockSpec((f, h), lambda i: (0, 0)),
                  pl.BlockSpec((f, h), lambda i: (0, 0))],
        out_specs=[pl.BlockSpec((bn, h), lambda i: (i, 0)),
                   pl.BlockSpec((bn, h), lambda i: (i, 0))],
        out_shape=[jax.ShapeDtypeStruct((n, h), jnp.float32),
                   jax.ShapeDtypeStruct((n, h), jnp.float32)],
    )(agg, r, bias.reshape(1, f), w1, w2)


def _pool_body(agg_ref, r_ref, b_ref, batch_ref, sums_ref, cnt_ref):
    i = pl.program_id(0)
    node = jnp.maximum(agg_ref[...] + r_ref[...] + b_ref[...], 0.0)
    bvec = batch_ref[...]
    bn = bvec.shape[0]
    gids = lax.broadcasted_iota(jnp.int32, (bn, _G), 1)
    onehot = (bvec == gids).astype(jnp.float32)
    dn = (((0,), (0,)), ((), ()))
    psum = lax.dot_general(onehot, node, dimension_numbers=dn,
                           preferred_element_type=jnp.float32)
    pcnt = lax.dot_general(onehot, jnp.ones((bn, 8), jnp.float32),
                           dimension_numbers=dn,
                           preferred_element_type=jnp.float32)

    @pl.when(i == 0)
    def _():
        sums_ref[...] = jnp.zeros_like(sums_ref)
        cnt_ref[...] = jnp.zeros_like(cnt_ref)

    sums_ref[...] += psum
    cnt_ref[...] += pcnt


def _pool(agg, r, bias, batch_2d, bn=2000):
    n, f = agg.shape
    return pl.pallas_call(
        _pool_body,
        grid=(n // bn,),
        in_specs=[pl.BlockSpec((bn, f), lambda i: (i, 0)),
                  pl.BlockSpec((bn, f), lambda i: (i, 0)),
                  pl.BlockSpec((1, f), lambda i: (0, 0)),
                  pl.BlockSpec((bn, 1), lambda i: (i, 0))],
        out_specs=[pl.BlockSpec((_G, f), lambda i: (0, 0)),
                   pl.BlockSpec((_G, 8), lambda i: (0, 0))],
        out_shape=[jax.ShapeDtypeStruct((_G, f), jnp.float32),
                   jax.ShapeDtypeStruct((_G, 8), jnp.float32)],
    )(agg, r, bias.reshape(1, f), batch_2d)


def _head_body(sums_ref, cnt_ref, w1_ref, b1_ref, w2_ref, b2_ref,
               gx_ref, fn_ref):
    cnt = jnp.maximum(cnt_ref[...][:, 0:1], 1.0)
    gx = sums_ref[...] / cnt
    gx_ref[...] = gx
    hh = jnp.maximum(
        jnp.dot(gx, w1_ref[...], preferred_element_type=jnp.float32)
        + b1_ref[...], 0.0)
    fc = (jnp.dot(hh, w2_ref[...], preferred_element_type=jnp.float32)
          + b2_ref[...])
    nrm = jnp.sqrt(jnp.sum(fc * fc, axis=1, keepdims=True))
    fn_ref[...] = fc / jnp.maximum(nrm, 1e-12)


def _head(sums, cnt, w1, b1, w2, b2):
    f = sums.shape[1]
    h2 = w1.shape[1]
    fo = w2.shape[1]
    return pl.pallas_call(
        _head_body,
        out_shape=[jax.ShapeDtypeStruct((_G, f), jnp.float32),
                   jax.ShapeDtypeStruct((_G, fo), jnp.float32)],
    )(sums, cnt, w1, b1.reshape(1, h2), w2, b2.reshape(1, fo))


def _gcn_norm(edge_index, edge_weight, num_nodes):
    row, col = edge_index[0], edge_index[1]
    deg = jnp.zeros((num_nodes,), dtype=edge_weight.dtype).at[col].add(edge_weight)
    deg_safe = jnp.where(deg > 0, deg, 1.0)
    dinv = jnp.where(deg > 0, deg_safe ** -0.5, 0.0)
    return dinv[row] * edge_weight * dinv[col]


def _aggregate(out, edge_index, norm):
    row, col = edge_index[0], edge_index[1]
    msg = out[row] * norm[:, None]
    return jnp.zeros_like(out).at[col].add(msg)


def kernel(x, edge_index, edge_attr, batch,
           init_w0, root_w0, bias0, init_w1, root_w1, bias1,
           head_w1, head_b1, head_w2, head_b2):
    n = x.shape[0]
    edge_weight = edge_attr.reshape(-1)
    norm = _gcn_norm(edge_index, edge_weight, n)

    out0, r0 = _mm2(x, init_w0, root_w0)
    agg0 = _aggregate(out0, edge_index, norm)

    out1, r1 = _fused_mm2(agg0, r0, bias0, init_w1, root_w1)
    agg1 = _aggregate(out1, edge_index, norm)

    sums, cnt = _pool(agg1, r1, bias1, batch.reshape(n, 1))
    graph_x, feat_n = _head(sums, cnt, head_w1, head_b1, head_w2, head_b2)
    return (graph_x, feat_n)


# speedup vs baseline: 1.6383x; 1.5739x over previous
import functools

import jax
import jax.numpy as jnp
from jax import lax
from jax.experimental import pallas as pl
from jax.experimental.pallas import tpu as pltpu
from jax.experimental.pallas import tpu_sc as plsc

_G = 128
_N = 10000
_E = 160000
_HID = 512

_NC = 2
_NS = 16
_NW = _NC * _NS
_EW = _E // _NW
_EWP = 5008
_NR = 8
_RS = 1280
_NPAD = _NR * _RS
_B = 48
_HCAP = 2560
_CAP = 2 * _HCAP
_W = 160
_GH0 = 157
_GH1 = 156
_ACC = _RS + 16

_mesh = functools.partial(
    plsc.VectorSubcoreMesh, core_axis_name="c", subcore_axis_name="s")
_sc_params = pltpu.CompilerParams(needs_layout_passes=False)


def _sc_bin_body(row_hbm, col_hbm, w_hbm,
                 brow_hbm, bdst_hbm, bw_hbm, cnt_hbm, deg_hbm,
                 rbuf, cbuf, wbuf, lrow, ldst, lw, degv, cntv):
    c = lax.axis_index("c")
    s = lax.axis_index("s")
    wid = s * _NC + c

    zero16f = jnp.zeros((16,), jnp.float32)
    zero16i = jnp.zeros((16,), jnp.int32)
    lanes = lax.broadcasted_iota(jnp.int32, (16,), 0)

    rbuf[pl.ds(_EWP - 16, 16)] = zero16i
    cbuf[pl.ds(_EWP - 16, 16)] = zero16i
    wbuf[pl.ds(_EWP - 16, 16)] = zero16f

    base_e = wid * _EW
    pltpu.sync_copy(row_hbm.at[pl.ds(base_e, _EW)], rbuf.at[pl.ds(0, _EW)])
    pltpu.sync_copy(col_hbm.at[pl.ds(base_e, _EW)], cbuf.at[pl.ds(0, _EW)])
    pltpu.sync_copy(w_hbm.at[pl.ds(base_e, _EW)], wbuf.at[pl.ds(0, _EW)])

    def dz_body(i, _):
        degv[pl.ds(i * 16, 16)] = zero16f
        return 0
    lax.fori_loop(0, _NPAD // 16, dz_body, 0)

    pad_row = jnp.broadcast_to((wid * 311) % _N, (16,)).astype(jnp.int32)
    pad_dst = jnp.broadcast_to(_RS + (wid % 16), (16,)).astype(jnp.int32)

    def prefill():
        def pre_body(i, _):
            o = i * 16
            lrow[pl.ds(o, 16)] = pad_row
            ldst[pl.ds(o, 16)] = pad_dst
            lw[pl.ds(o, 16)] = zero16f
            return 0
        lax.fori_loop(0, _NR * (_HCAP // 16), pre_body, 0)

    def scan_body(g, ks):
        o = g * 16
        colv = cbuf[pl.ds(o, 16)]
        rowv = rbuf[pl.ds(o, 16)]
        wv = wbuf[pl.ds(o, 16)]
        plsc.addupdate_scatter(degv, [colv], wv)
        out = []
        for r in range(_NR):
            m = (colv >= r * _RS) & (colv < (r + 1) * _RS)
            k = ks[r]
            plsc.store_compressed(lrow.at[pl.ds(r * _HCAP + k, 16)],
                                  rowv, mask=m)
            plsc.store_compressed(ldst.at[pl.ds(r * _HCAP + k, 16)],
                                  colv - r * _RS, mask=m)
            plsc.store_compressed(lw.at[pl.ds(r * _HCAP + k, 16)],
                                  wv, mask=m)
            out.append(k + jnp.sum(m.astype(jnp.int32)))
        return tuple(out)

    nbv = zero16i
    for h, (g0, g1) in enumerate(((0, _GH0), (_GH0, _GH0 + _GH1))):
        prefill()
        ks = lax.fori_loop(g0, g1, scan_body, (0,) * _NR)
        for r in range(_NR):
            nbv = jnp.where(lanes == r * 2 + h, ks[r], nbv)
            pltpu.sync_copy(
                lrow.at[pl.ds(r * _HCAP, _HCAP)],
                brow_hbm.at[r, wid, pl.ds(h * _HCAP, _HCAP)])
            pltpu.sync_copy(
                ldst.at[pl.ds(r * _HCAP, _HCAP)],
                bdst_hbm.at[r, wid, pl.ds(h * _HCAP, _HCAP)])
            pltpu.sync_copy(
                lw.at[pl.ds(r * _HCAP, _HCAP)],
                bw_hbm.at[r, wid, pl.ds(h * _HCAP, _HCAP)])

    cntv[...] = nbv
    pltpu.sync_copy(degv, deg_hbm.at[wid])
    pltpu.sync_copy(cntv, cnt_hbm.at[wid])


def _sc_bin(row, col, w):
    kfn = pl.kernel(
        _sc_bin_body,
        mesh=_mesh(),
        compiler_params=_sc_params,
        out_type=[
            jax.ShapeDtypeStruct((_NR, _NW, _CAP), jnp.int32),
            jax.ShapeDtypeStruct((_NR, _NW, _CAP), jnp.int32),
            jax.ShapeDtypeStruct((_NR, _NW, _CAP), jnp.float32),
            jax.ShapeDtypeStruct((_NW, 16), jnp.int32),
            jax.ShapeDtypeStruct((_NW, _NPAD), jnp.float32),
        ],
        scratch_types=[
            pltpu.VMEM((_EWP,), jnp.int32),
            pltpu.VMEM((_EWP,), jnp.int32),
            pltpu.VMEM((_EWP,), jnp.float32),
            pltpu.VMEM((_NR * _HCAP,), jnp.int32),
            pltpu.VMEM((_NR * _HCAP,), jnp.int32),
            pltpu.VMEM((_NR * _HCAP,), jnp.float32),
            pltpu.VMEM((_NPAD,), jnp.float32),
            pltpu.VMEM((16,), jnp.int32),
        ],
    )
    return kfn(row, col, w)


def _sc_norm_body(brow_hbm, bdst_hbm, bw_hbm, dinv_hbm,
                  bnorm_hbm,
                  srow, sdst, sw, snorm, dinv):
    c = lax.axis_index("c")
    s = lax.axis_index("s")
    wid = s * _NC + c

    pltpu.sync_copy(dinv_hbm, dinv)
    for r in range(_NR):
        pltpu.sync_copy(brow_hbm.at[r, wid], srow)
        pltpu.sync_copy(bdst_hbm.at[r, wid], sdst)
        pltpu.sync_copy(bw_hbm.at[r, wid], sw)

        def nb_body(i, _):
            o = i * 16
            rv = srow[pl.ds(o, 16)]
            dv = sdst[pl.ds(o, 16)]
            wv = sw[pl.ds(o, 16)]
            di_r = plsc.load_gather(dinv, [rv])
            ci = jnp.minimum(dv + r * _RS, _NPAD - 1)
            di_c = plsc.load_gather(dinv, [ci])
            snorm[pl.ds(o, 16)] = di_r * wv * di_c
            return 0
        lax.fori_loop(0, _CAP // 16, nb_body, 0)
        pltpu.sync_copy(snorm, bnorm_hbm.at[r, wid])


def _sc_norm(brow, bdst, bw, dinv):
    kfn = pl.kernel(
        _sc_norm_body,
        mesh=_mesh(),
        compiler_params=_sc_params,
        out_type=jax.ShapeDtypeStruct((_NR, _NW, _CAP), jnp.float32),
        scratch_types=[
            pltpu.VMEM((_CAP,), jnp.int32),
            pltpu.VMEM((_CAP,), jnp.int32),
            pltpu.VMEM((_CAP,), jnp.float32),
            pltpu.VMEM((_CAP,), jnp.float32),
            pltpu.VMEM((_NPAD,), jnp.float32),
        ],
    )
    return kfn(brow, bdst, bw, dinv)


def _sc_agg_body(out_hbm, brow_hbm, bdst_hbm, bnorm_hbm, cnt_hbm,
                 agg_hbm,
                 srow, sdst, snorm, crow, cdst, cnorm, acc, rowbuf, cntv,
                 gsem):
    c = lax.axis_index("c")
    s = lax.axis_index("s")

    zero16f = jnp.zeros((16,), jnp.float32)
    zero16i = jnp.zeros((16,), jnp.int32)
    lanes = lax.broadcasted_iota(jnp.int32, (16,), 0)
    pad_row = jnp.broadcast_to((s * 617) % _N, (16,)).astype(jnp.int32)

    for rd in range(2):
        r = 4 * c + 2 * rd + s // 8
        lo = (s % 8) * _W

        def az_body(i, _):
            acc[i // 32, pl.ds((i % 32) * 16, 16)] = zero16f
            return 0
        lax.fori_loop(0, _W * 32, az_body, 0)

        def seg_body(sg, _):
            p = sg // 2
            h = sg % 2
            pltpu.sync_copy(cnt_hbm.at[p], cntv)
            kseg = jnp.sum(jnp.where(lanes == r * 2 + h, cntv[...], 0))
            pltpu.sync_copy(brow_hbm.at[r, p, pl.ds(h * _HCAP, _HCAP)], srow)
            pltpu.sync_copy(bdst_hbm.at[r, p, pl.ds(h * _HCAP, _HCAP)], sdst)
            pltpu.sync_copy(bnorm_hbm.at[r, p, pl.ds(h * _HCAP, _HCAP)], snorm)

            def scan_b(g, k):
                o = g * 16
                dv = sdst[pl.ds(o, 16)]
                m = (dv >= lo) & (dv < lo + _W)
                plsc.store_compressed(crow.at[pl.ds(k, 16)],
                                      srow[pl.ds(o, 16)], mask=m)
                plsc.store_compressed(cdst.at[pl.ds(k, 16)], dv - lo, mask=m)
                plsc.store_compressed(cnorm.at[pl.ds(k, 16)],
                                      snorm[pl.ds(o, 16)], mask=m)
                return k + jnp.sum(m.astype(jnp.int32))

            k = lax.fori_loop(0, (kseg + 15) // 16, scan_b, 0)

            for u in range(_B // 16):
                crow[pl.ds(k + u * 16, 16)] = pad_row
                cdst[pl.ds(k + u * 16, 16)] = zero16i
                cnorm[pl.ds(k + u * 16, 16)] = zero16f

            def batch_b(j, _):
                idx = crow.at[pl.ds(j * _B, _B)]
                pltpu.async_copy(out_hbm.at[idx], rowbuf, gsem).wait()

                def edge_b(jj, _):
                    ii = jnp.broadcast_to(j * _B + jj, (16,)).astype(jnp.int32)
                    nsp = plsc.load_gather(cnorm, [ii])
                    dsp = plsc.load_gather(cdst, [ii])
                    for cc in range(32):
                        v = rowbuf[jj, pl.ds(cc * 16, 16)]
                        plsc.addupdate_scatter(
                            acc, [dsp, lanes + cc * 16], v * nsp)
                    return 0
                lax.fori_loop(0, _B, edge_b, 0)
                return 0

            lax.fori_loop(0, (k + _B - 1) // _B, batch_b, 0)
            return 0

        lax.fori_loop(0, 2 * _NW, seg_body, 0)

        pltpu.sync_copy(acc, agg_hbm.at[pl.ds(r * _RS + lo, _W)])


def _sc_agg(out_mat, brow, bdst, bnorm, cnt):
    kfn = pl.kernel(
        _sc_agg_body,
        mesh=_mesh(),
        compiler_params=_sc_params,
        out_type=jax.ShapeDtypeStruct((_NPAD, _HID), jnp.float32),
        scratch_types=[
            pltpu.VMEM((_HCAP,), jnp.int32),
            pltpu.VMEM((_HCAP,), jnp.int32),
            pltpu.VMEM((_HCAP,), jnp.float32),
            pltpu.VMEM((_HCAP + _B,), jnp.int32),
            pltpu.VMEM((_HCAP + _B,), jnp.int32),
            pltpu.VMEM((_HCAP + _B,), jnp.float32),
            pltpu.VMEM((_W, _HID), jnp.float32),
            pltpu.VMEM((_B, _HID), jnp.float32),
            pltpu.VMEM((16,), jnp.int32),
            pltpu.SemaphoreType.DMA,
        ],
    )
    return kfn(out_mat, brow, bdst, bnorm, cnt)


def _mm2_body(x_ref, w1_ref, w2_ref, o1_ref, o2_ref):
    x = x_ref[...]
    o1_ref[...] = jnp.dot(x, w1_ref[...], preferred_element_type=jnp.float32)
    o2_ref[...] = jnp.dot(x, w2_ref[...], preferred_element_type=jnp.float32)


def _mm2(x, w1, w2, bn=2000):
    n, f = x.shape
    h = w1.shape[1]
    return pl.pallas_call(
        _mm2_body,
        grid=(_N // bn,),
        in_specs=[pl.BlockSpec((bn, f), lambda i: (i, 0)),
                  pl.BlockSpec((f, h), lambda i: (0, 0)),
                  pl.BlockSpec((f, h), lambda i: (0, 0))],
        out_specs=[pl.BlockSpec((bn, h), lambda i: (i, 0)),
                   pl.BlockSpec((bn, h), lambda i: (i, 0))],
        out_shape=[jax.ShapeDtypeStruct((_N, h), jnp.float32),
                   jax.ShapeDtypeStruct((_N, h), jnp.float32)],
    )(x, w1, w2)


def _dinv_body(dp_ref, dinv_ref):
    deg = jnp.sum(dp_ref[...], axis=0)
    safe = jnp.where(deg > 0, deg, 1.0)
    dinv_ref[...] = jnp.where(deg > 0, lax.rsqrt(safe), 0.0)


def _dinv(deg_part):
    dp = deg_part.reshape(_NW, _NPAD // 128, 128)
    out = pl.pallas_call(
        _dinv_body,
        out_shape=jax.ShapeDtypeStruct((_NPAD // 128, 128), jnp.float32),
    )(dp)
    return out.reshape(_NPAD)


def _fused_mm2_body(agg_ref, r_ref, b_ref, w1_ref, w2_ref, o1_ref, o2_ref):
    h = jnp.maximum(agg_ref[...] + r_ref[...] + b_ref[...], 0.0)
    o1_ref[...] = jnp.dot(h, w1_ref[...], preferred_element_type=jnp.float32)
    o2_ref[...] = jnp.dot(h, w2_ref[...], preferred_element_type=jnp.float32)


def _fused_mm2(agg, r, bias, w1, w2, bn=2000):
    f = agg.shape[1]
    h = w1.shape[1]
    return pl.pallas_call(
        _fused_mm2_body,
        grid=(_N // bn,),
        in_specs=[pl.BlockSpec((bn, f), lambda i: (i, 0)),
                  pl.BlockSpec((bn, f), lambda i: (i, 0)),
                  pl.BlockSpec((1, f), lambda i: (0, 0)),
                  pl.BlockSpec((f, h), lambda i: (0, 0)),
                  pl.BlockSpec((f, h), lambda i: (0, 0))],
        out_specs=[pl.BlockSpec((bn, h), lambda i: (i, 0)),
                   pl.BlockSpec((bn, h), lambda i: (i, 0))],
        out_shape=[jax.ShapeDtypeStruct((_N, h), jnp.float32),
                   jax.ShapeDtypeStruct((_N, h), jnp.float32)],
    )(agg, r, bias.reshape(1, f), w1, w2)


def _pool_body(agg_ref, r_ref, b_ref, batch_ref, sums_ref, cnt_ref):
    i = pl.program_id(0)
    node = jnp.maximum(agg_ref[...] + r_ref[...] + b_ref[...], 0.0)
    bvec = batch_ref[...]
    bn = bvec.shape[0]
    gids = lax.broadcasted_iota(jnp.int32, (bn, _G), 1)
    onehot = (bvec == gids).astype(jnp.float32)
    dn = (((0,), (0,)), ((), ()))
    psum = lax.dot_general(onehot, node, dimension_numbers=dn,
                           preferred_element_type=jnp.float32)
    pcnt = lax.dot_general(onehot, jnp.ones((bn, 8), jnp.float32),
                           dimension_numbers=dn,
                           preferred_element_type=jnp.float32)

    @pl.when(i == 0)
    def _():
        sums_ref[...] = jnp.zeros_like(sums_ref)
        cnt_ref[...] = jnp.zeros_like(cnt_ref)

    sums_ref[...] += psum
    cnt_ref[...] += pcnt


def _pool(agg, r, bias, batch_2d, bn=2000):
    f = agg.shape[1]
    return pl.pallas_call(
        _pool_body,
        grid=(_N // bn,),
        in_specs=[pl.BlockSpec((bn, f), lambda i: (i, 0)),
                  pl.BlockSpec((bn, f), lambda i: (i, 0)),
                  pl.BlockSpec((1, f), lambda i: (0, 0)),
                  pl.BlockSpec((bn, 1), lambda i: (i, 0))],
        out_specs=[pl.BlockSpec((_G, f), lambda i: (0, 0)),
                   pl.BlockSpec((_G, 8), lambda i: (0, 0))],
        out_shape=[jax.ShapeDtypeStruct((_G, f), jnp.float32),
                   jax.ShapeDtypeStruct((_G, 8), jnp.float32)],
    )(agg, r, bias.reshape(1, f), batch_2d)


def _head_body(sums_ref, cnt_ref, w1_ref, b1_ref, w2_ref, b2_ref,
               gx_ref, fn_ref):
    cnt = jnp.maximum(cnt_ref[...][:, 0:1], 1.0)
    gx = sums_ref[...] / cnt
    gx_ref[...] = gx
    hh = jnp.maximum(
        jnp.dot(gx, w1_ref[...], preferred_element_type=jnp.float32)
        + b1_ref[...], 0.0)
    fc = (jnp.dot(hh, w2_ref[...], preferred_element_type=jnp.float32)
          + b2_ref[...])
    nrm = jnp.sqrt(jnp.sum(fc * fc, axis=1, keepdims=True))
    fn_ref[...] = fc / jnp.maximum(nrm, 1e-12)


def _head(sums, cnt, w1, b1, w2, b2):
    f = sums.shape[1]
    h2 = w1.shape[1]
    fo = w2.shape[1]
    return pl.pallas_call(
        _head_body,
        out_shape=[jax.ShapeDtypeStruct((_G, f), jnp.float32),
                   jax.ShapeDtypeStruct((_G, fo), jnp.float32)],
    )(sums, cnt, w1, b1.reshape(1, h2), w2, b2.reshape(1, fo))


def kernel(x, edge_index, edge_attr, batch,
           init_w0, root_w0, bias0, init_w1, root_w1, bias1,
           head_w1, head_b1, head_w2, head_b2):
    row = edge_index[0]
    col = edge_index[1]
    w = edge_attr.reshape(-1)

    brow, bdst, bw, cnt, deg_part = _sc_bin(row, col, w)
    dinv = _dinv(deg_part)
    bnorm = _sc_norm(brow, bdst, bw, dinv)
    out0, r0 = _mm2(x, init_w0, root_w0)
    agg0 = _sc_agg(out0, brow, bdst, bnorm, cnt)

    out1, r1 = _fused_mm2(agg0, r0, bias0, init_w1, root_w1)
    agg1 = _sc_agg(out1, brow, bdst, bnorm, cnt)

    sums, cnt_g = _pool(agg1, r1, bias1, batch.reshape(_N, 1))
    graph_x, feat_n = _head(sums, cnt_g, head_w1, head_b1, head_w2, head_b2)
    return (graph_x, feat_n)


# speedup vs baseline: 4.3830x; 2.6754x over previous
import functools

import jax
import jax.numpy as jnp
from jax import lax
from jax.experimental import pallas as pl
from jax.experimental.pallas import tpu as pltpu
from jax.experimental.pallas import tpu_sc as plsc

_G = 128
_N = 10000
_E = 160000
_HID = 512

_NC = 2
_NS = 16
_NW = _NC * _NS
_EW = _E // _NW
_EWP = 5008
_NR = 8
_RS = 1280
_NPAD = _NR * _RS
_B = 48
_HCAP = 2560
_CAP = 2 * _HCAP
_W = 160
_GH0 = 157
_GH1 = 156
_ACC = _RS + 16

_mesh = functools.partial(
    plsc.VectorSubcoreMesh, core_axis_name="c", subcore_axis_name="s")
_sc_params = pltpu.CompilerParams(needs_layout_passes=False)


def _sc_bin_body(row_hbm, col_hbm, w_hbm,
                 brow_hbm, bdst_hbm, bw_hbm, cnt_hbm, deg_hbm,
                 rbuf, cbuf, wbuf, lrow, ldst, lw, degv, cntv):
    c = lax.axis_index("c")
    s = lax.axis_index("s")
    wid = s * _NC + c

    zero16f = jnp.zeros((16,), jnp.float32)
    zero16i = jnp.zeros((16,), jnp.int32)
    lanes = lax.broadcasted_iota(jnp.int32, (16,), 0)

    rbuf[pl.ds(_EWP - 16, 16)] = zero16i
    cbuf[pl.ds(_EWP - 16, 16)] = zero16i
    wbuf[pl.ds(_EWP - 16, 16)] = zero16f

    base_e = wid * _EW
    pltpu.sync_copy(row_hbm.at[pl.ds(base_e, _EW)], rbuf.at[pl.ds(0, _EW)])
    pltpu.sync_copy(col_hbm.at[pl.ds(base_e, _EW)], cbuf.at[pl.ds(0, _EW)])
    pltpu.sync_copy(w_hbm.at[pl.ds(base_e, _EW)], wbuf.at[pl.ds(0, _EW)])

    def dz_body(i, _):
        degv[pl.ds(i * 16, 16)] = zero16f
        return 0
    lax.fori_loop(0, _NPAD // 16, dz_body, 0)

    pad_row = jnp.broadcast_to((wid * 311) % _N, (16,)).astype(jnp.int32)
    pad_dst = jnp.broadcast_to(_RS + (wid % 16), (16,)).astype(jnp.int32)

    def prefill():
        def pre_body(i, _):
            o = i * 16
            lrow[pl.ds(o, 16)] = pad_row
            ldst[pl.ds(o, 16)] = pad_dst
            lw[pl.ds(o, 16)] = zero16f
            return 0
        lax.fori_loop(0, _NR * (_HCAP // 16), pre_body, 0)

    def scan_body(g, ks):
        o = g * 16
        colv = cbuf[pl.ds(o, 16)]
        rowv = rbuf[pl.ds(o, 16)]
        wv = wbuf[pl.ds(o, 16)]
        plsc.addupdate_scatter(degv, [colv], wv)
        out = []
        for r in range(_NR):
            m = (colv >= r * _RS) & (colv < (r + 1) * _RS)
            k = ks[r]
            plsc.store_compressed(lrow.at[pl.ds(r * _HCAP + k, 16)],
                                  rowv, mask=m)
            plsc.store_compressed(ldst.at[pl.ds(r * _HCAP + k, 16)],
                                  colv - r * _RS, mask=m)
            plsc.store_compressed(lw.at[pl.ds(r * _HCAP + k, 16)],
                                  wv, mask=m)
            out.append(k + jnp.sum(m.astype(jnp.int32)))
        return tuple(out)

    nbv = zero16i
    for h, (g0, g1) in enumerate(((0, _GH0), (_GH0, _GH0 + _GH1))):
        prefill()
        ks = lax.fori_loop(g0, g1, scan_body, (0,) * _NR)
        for r in range(_NR):
            nbv = jnp.where(lanes == r * 2 + h, ks[r], nbv)
            pltpu.sync_copy(
                lrow.at[pl.ds(r * _HCAP, _HCAP)],
                brow_hbm.at[r, wid, pl.ds(h * _HCAP, _HCAP)])
            pltpu.sync_copy(
                ldst.at[pl.ds(r * _HCAP, _HCAP)],
                bdst_hbm.at[r, wid, pl.ds(h * _HCAP, _HCAP)])
            pltpu.sync_copy(
                lw.at[pl.ds(r * _HCAP, _HCAP)],
                bw_hbm.at[r, wid, pl.ds(h * _HCAP, _HCAP)])

    cntv[...] = nbv
    pltpu.sync_copy(degv, deg_hbm.at[wid])
    pltpu.sync_copy(cntv, cnt_hbm.at[wid])


def _sc_bin(row, col, w):
    kfn = pl.kernel(
        _sc_bin_body,
        mesh=_mesh(),
        compiler_params=_sc_params,
        out_type=[
            jax.ShapeDtypeStruct((_NR, _NW, _CAP), jnp.int32),
            jax.ShapeDtypeStruct((_NR, _NW, _CAP), jnp.int32),
            jax.ShapeDtypeStruct((_NR, _NW, _CAP), jnp.float32),
            jax.ShapeDtypeStruct((_NW, 16), jnp.int32),
            jax.ShapeDtypeStruct((_NW, _NPAD), jnp.float32),
        ],
        scratch_types=[
            pltpu.VMEM((_EWP,), jnp.int32),
            pltpu.VMEM((_EWP,), jnp.int32),
            pltpu.VMEM((_EWP,), jnp.float32),
            pltpu.VMEM((_NR * _HCAP,), jnp.int32),
            pltpu.VMEM((_NR * _HCAP,), jnp.int32),
            pltpu.VMEM((_NR * _HCAP,), jnp.float32),
            pltpu.VMEM((_NPAD,), jnp.float32),
            pltpu.VMEM((16,), jnp.int32),
        ],
    )
    return kfn(row, col, w)


def _sc_norm_body(brow_hbm, bdst_hbm, bw_hbm, dinv_hbm,
                  bnorm_hbm,
                  srow, sdst, sw, snorm, dinv):
    c = lax.axis_index("c")
    s = lax.axis_index("s")
    wid = s * _NC + c

    pltpu.sync_copy(dinv_hbm, dinv)
    for r in range(_NR):
        pltpu.sync_copy(brow_hbm.at[r, wid], srow)
        pltpu.sync_copy(bdst_hbm.at[r, wid], sdst)
        pltpu.sync_copy(bw_hbm.at[r, wid], sw)

        def nb_body(i, _):
            o = i * 16
            rv = srow[pl.ds(o, 16)]
            dv = sdst[pl.ds(o, 16)]
            wv = sw[pl.ds(o, 16)]
            di_r = plsc.load_gather(dinv, [rv])
            ci = jnp.minimum(dv + r * _RS, _NPAD - 1)
            di_c = plsc.load_gather(dinv, [ci])
            snorm[pl.ds(o, 16)] = di_r * wv * di_c
            return 0
        lax.fori_loop(0, _CAP // 16, nb_body, 0)
        pltpu.sync_copy(snorm, bnorm_hbm.at[r, wid])


def _sc_norm(brow, bdst, bw, dinv):
    kfn = pl.kernel(
        _sc_norm_body,
        mesh=_mesh(),
        compiler_params=_sc_params,
        out_type=jax.ShapeDtypeStruct((_NR, _NW, _CAP), jnp.float32),
        scratch_types=[
            pltpu.VMEM((_CAP,), jnp.int32),
            pltpu.VMEM((_CAP,), jnp.int32),
            pltpu.VMEM((_CAP,), jnp.float32),
            pltpu.VMEM((_CAP,), jnp.float32),
            pltpu.VMEM((_NPAD,), jnp.float32),
        ],
    )
    return kfn(brow, bdst, bw, dinv)


def _sc_agg_body(out_hbm, brow_hbm, bdst_hbm, bnorm_hbm, cnt_hbm,
                 aggf_hbm,
                 srow, sdst, snorm, crow, cdst, cnorm, acc, rowbuf, cntall,
                 gsem, ssem):
    c = lax.axis_index("c")
    s = lax.axis_index("s")

    zero16f = jnp.zeros((16,), jnp.float32)
    zero16i = jnp.zeros((16,), jnp.int32)
    lanes = lax.broadcasted_iota(jnp.int32, (16,), 0)
    pad_row = jnp.broadcast_to((s * 617) % _N, (16,)).astype(jnp.int32)

    pltpu.sync_copy(cnt_hbm, cntall)

    for rd in range(2):
        r = 4 * c + 2 * rd + s // 8
        lo = (s % 8) * _W

        @plsc.parallel_loop(0, _W * 32, unroll=4)
        def _(i):
            acc[pl.ds(i * 16, 16)] = zero16f

        def make_batch_body():
            def batch_b(j, _):
                idx = crow.at[pl.ds(j * _B, _B)]
                pltpu.async_copy(out_hbm.at[idx], rowbuf, gsem).wait()

                @plsc.parallel_loop(0, _B, unroll=2)
                def _(jj):
                    ii = jnp.broadcast_to(j * _B + jj, (16,)).astype(jnp.int32)
                    nsp = plsc.load_gather(cnorm, [ii])
                    dsp = plsc.load_gather(cdst, [ii])
                    base = dsp * _HID + lanes
                    for cc in range(32):
                        v = rowbuf[jj, pl.ds(cc * 16, 16)]
                        plsc.addupdate_scatter(acc, [base + cc * 16], v * nsp)
                return 0
            return batch_b

        batch_b = make_batch_body()

        def seg_body(sg, kin):
            p = sg // 2
            h = sg % 2
            cv = cntall[pl.ds(p * 16, 16)]
            kseg = jnp.sum(jnp.where(lanes == r * 2 + h, cv, 0))
            cp1 = pltpu.async_copy(
                brow_hbm.at[r, p, pl.ds(h * _HCAP, _HCAP)], srow, ssem)
            cp2 = pltpu.async_copy(
                bdst_hbm.at[r, p, pl.ds(h * _HCAP, _HCAP)], sdst, ssem)
            cp3 = pltpu.async_copy(
                bnorm_hbm.at[r, p, pl.ds(h * _HCAP, _HCAP)], snorm, ssem)
            cp1.wait()
            cp2.wait()
            cp3.wait()

            def scan_b(g, k):
                o = g * 16
                dv = sdst[pl.ds(o, 16)]
                m = (dv >= lo) & (dv < lo + _W)
                plsc.store_compressed(crow.at[pl.ds(k, 16)],
                                      srow[pl.ds(o, 16)], mask=m)
                plsc.store_compressed(cdst.at[pl.ds(k, 16)], dv - lo, mask=m)
                plsc.store_compressed(cnorm.at[pl.ds(k, 16)],
                                      snorm[pl.ds(o, 16)], mask=m)
                return k + jnp.sum(m.astype(jnp.int32))

            kcur = lax.fori_loop(0, (kseg + 15) // 16, scan_b, kin)

            nbf = kcur // _B
            lax.fori_loop(0, nbf, batch_b, 0)
            for u in range(_B // 16):
                o = u * 16
                crow[pl.ds(o, 16)] = crow[pl.ds(nbf * _B + o, 16)]
                cdst[pl.ds(o, 16)] = cdst[pl.ds(nbf * _B + o, 16)]
                cnorm[pl.ds(o, 16)] = cnorm[pl.ds(nbf * _B + o, 16)]
            return kcur - nbf * _B

        kleft = lax.fori_loop(0, 2 * _NW, seg_body, 0)

        for u in range(_B // 16):
            o = u * 16
            crow[pl.ds(kleft + o, 16)] = pad_row
            cdst[pl.ds(kleft + o, 16)] = zero16i
            cnorm[pl.ds(kleft + o, 16)] = zero16f
        lax.fori_loop(0, (kleft + _B - 1) // _B, batch_b, 0)

        pltpu.sync_copy(
            acc, aggf_hbm.at[pl.ds((r * _RS + lo) * _HID, _W * _HID)])


def _sc_agg(out_mat, brow, bdst, bnorm, cnt):
    kfn = pl.kernel(
        _sc_agg_body,
        mesh=_mesh(),
        compiler_params=_sc_params,
        out_type=jax.ShapeDtypeStruct((_NPAD * _HID,), jnp.float32),
        scratch_types=[
            pltpu.VMEM((_HCAP,), jnp.int32),
            pltpu.VMEM((_HCAP,), jnp.int32),
            pltpu.VMEM((_HCAP,), jnp.float32),
            pltpu.VMEM((_HCAP + 112,), jnp.int32),
            pltpu.VMEM((_HCAP + 112,), jnp.int32),
            pltpu.VMEM((_HCAP + 112,), jnp.float32),
            pltpu.VMEM((_W * _HID,), jnp.float32),
            pltpu.VMEM((_B, _HID), jnp.float32),
            pltpu.VMEM((_NW * 16,), jnp.int32),
            pltpu.SemaphoreType.DMA,
            pltpu.SemaphoreType.DMA,
        ],
    )
    return kfn(out_mat, brow, bdst, bnorm,
               cnt.reshape(-1)).reshape(_NPAD, _HID)


def _mm2_body(x_ref, w1_ref, w2_ref, o1_ref, o2_ref):
    x = x_ref[...]
    o1_ref[...] = jnp.dot(x, w1_ref[...], preferred_element_type=jnp.float32)
    o2_ref[...] = jnp.dot(x, w2_ref[...], preferred_element_type=jnp.float32)


def _mm2(x, w1, w2, bn=2000):
    n, f = x.shape
    h = w1.shape[1]
    return pl.pallas_call(
        _mm2_body,
        grid=(_N // bn,),
        in_specs=[pl.BlockSpec((bn, f), lambda i: (i, 0)),
                  pl.BlockSpec((f, h), lambda i: (0, 0)),
                  pl.BlockSpec((f, h), lambda i: (0, 0))],
        out_specs=[pl.BlockSpec((bn, h), lambda i: (i, 0)),
                   pl.BlockSpec((bn, h), lambda i: (i, 0))],
        out_shape=[jax.ShapeDtypeStruct((_N, h), jnp.float32),
                   jax.ShapeDtypeStruct((_N, h), jnp.float32)],
    )(x, w1, w2)


def _dinv_body(dp_ref, dinv_ref):
    deg = jnp.sum(dp_ref[...], axis=0)
    safe = jnp.where(deg > 0, deg, 1.0)
    dinv_ref[...] = jnp.where(deg > 0, lax.rsqrt(safe), 0.0)


def _dinv(deg_part):
    dp = deg_part.reshape(_NW, _NPAD // 128, 128)
    out = pl.pallas_call(
        _dinv_body,
        out_shape=jax.ShapeDtypeStruct((_NPAD // 128, 128), jnp.float32),
    )(dp)
    return out.reshape(_NPAD)


def _fused_mm2_body(agg_ref, r_ref, b_ref, w1_ref, w2_ref, o1_ref, o2_ref):
    h = jnp.maximum(agg_ref[...] + r_ref[...] + b_ref[...], 0.0)
    o1_ref[...] = jnp.dot(h, w1_ref[...], preferred_element_type=jnp.float32)
    o2_ref[...] = jnp.dot(h, w2_ref[...], preferred_element_type=jnp.float32)


def _fused_mm2(agg, r, bias, w1, w2, bn=2000):
    f = agg.shape[1]
    h = w1.shape[1]
    return pl.pallas_call(
        _fused_mm2_body,
        grid=(_N // bn,),
        in_specs=[pl.BlockSpec((bn, f), lambda i: (i, 0)),
                  pl.BlockSpec((bn, f), lambda i: (i, 0)),
                  pl.BlockSpec((1, f), lambda i: (0, 0)),
                  pl.BlockSpec((f, h), lambda i: (0, 0)),
                  pl.BlockSpec((f, h), lambda i: (0, 0))],
        out_specs=[pl.BlockSpec((bn, h), lambda i: (i, 0)),
                   pl.BlockSpec((bn, h), lambda i: (i, 0))],
        out_shape=[jax.ShapeDtypeStruct((_N, h), jnp.float32),
                   jax.ShapeDtypeStruct((_N, h), jnp.float32)],
    )(agg, r, bias.reshape(1, f), w1, w2)


def _pool_body(agg_ref, r_ref, b_ref, batch_ref, sums_ref, cnt_ref):
    i = pl.program_id(0)
    node = jnp.maximum(agg_ref[...] + r_ref[...] + b_ref[...], 0.0)
    bvec = batch_ref[...]
    bn = bvec.shape[0]
    gids = lax.broadcasted_iota(jnp.int32, (bn, _G), 1)
    onehot = (bvec == gids).astype(jnp.float32)
    dn = (((0,), (0,)), ((), ()))
    psum = lax.dot_general(onehot, node, dimension_numbers=dn,
                           preferred_element_type=jnp.float32)
    pcnt = lax.dot_general(onehot, jnp.ones((bn, 8), jnp.float32),
                           dimension_numbers=dn,
                           preferred_element_type=jnp.float32)

    @pl.when(i == 0)
    def _():
        sums_ref[...] = jnp.zeros_like(sums_ref)
        cnt_ref[...] = jnp.zeros_like(cnt_ref)

    sums_ref[...] += psum
    cnt_ref[...] += pcnt


def _pool(agg, r, bias, batch_2d, bn=2000):
    f = agg.shape[1]
    return pl.pallas_call(
        _pool_body,
        grid=(_N // bn,),
        in_specs=[pl.BlockSpec((bn, f), lambda i: (i, 0)),
                  pl.BlockSpec((bn, f), lambda i: (i, 0)),
                  pl.BlockSpec((1, f), lambda i: (0, 0)),
                  pl.BlockSpec((bn, 1), lambda i: (i, 0))],
        out_specs=[pl.BlockSpec((_G, f), lambda i: (0, 0)),
                   pl.BlockSpec((_G, 8), lambda i: (0, 0))],
        out_shape=[jax.ShapeDtypeStruct((_G, f), jnp.float32),
                   jax.ShapeDtypeStruct((_G, 8), jnp.float32)],
    )(agg, r, bias.reshape(1, f), batch_2d)


def _head_body(sums_ref, cnt_ref, w1_ref, b1_ref, w2_ref, b2_ref,
               gx_ref, fn_ref):
    cnt = jnp.maximum(cnt_ref[...][:, 0:1], 1.0)
    gx = sums_ref[...] / cnt
    gx_ref[...] = gx
    hh = jnp.maximum(
        jnp.dot(gx, w1_ref[...], preferred_element_type=jnp.float32)
        + b1_ref[...], 0.0)
    fc = (jnp.dot(hh, w2_ref[...], preferred_element_type=jnp.float32)
          + b2_ref[...])
    nrm = jnp.sqrt(jnp.sum(fc * fc, axis=1, keepdims=True))
    fn_ref[...] = fc / jnp.maximum(nrm, 1e-12)


def _head(sums, cnt, w1, b1, w2, b2):
    f = sums.shape[1]
    h2 = w1.shape[1]
    fo = w2.shape[1]
    return pl.pallas_call(
        _head_body,
        out_shape=[jax.ShapeDtypeStruct((_G, f), jnp.float32),
                   jax.ShapeDtypeStruct((_G, fo), jnp.float32)],
    )(sums, cnt, w1, b1.reshape(1, h2), w2, b2.reshape(1, fo))


def kernel(x, edge_index, edge_attr, batch,
           init_w0, root_w0, bias0, init_w1, root_w1, bias1,
           head_w1, head_b1, head_w2, head_b2):
    row = edge_index[0]
    col = edge_index[1]
    w = edge_attr.reshape(-1)

    brow, bdst, bw, cnt, deg_part = _sc_bin(row, col, w)
    dinv = _dinv(deg_part)
    bnorm = _sc_norm(brow, bdst, bw, dinv)
    out0, r0 = _mm2(x, init_w0, root_w0)
    agg0 = _sc_agg(out0, brow, bdst, bnorm, cnt)

    out1, r1 = _fused_mm2(agg0, r0, bias0, init_w1, root_w1)
    agg1 = _sc_agg(out1, brow, bdst, bnorm, cnt)

    sums, cnt_g = _pool(agg1, r1, bias1, batch.reshape(_N, 1))
    graph_x, feat_n = _head(sums, cnt_g, head_w1, head_b1, head_w2, head_b2)
    return (graph_x, feat_n)
